# Initial kernel scaffold; baseline (speedup 1.0000x reference)
import jax
import jax.numpy as jnp
from jax.experimental import pallas as pl


def kernel(predictions, targets):
    raise NotImplementedError("write your pallas kernel here")


# speedup vs baseline: 1.1572x; 1.1572x over previous
import jax
import jax.numpy as jnp
from jax.experimental import pallas as pl
from jax.experimental.pallas import tpu as pltpu

_M = 2
_R = 0.2
_EPS = 1e-8
_T = 1024
_N = _T - _M
_TPAD = 1032
_BR = 256


def _entropy_kernel(rows_ref, cols_ref, out_ref):
    rows = rows_ref[0]
    x = rows[0:1, :]
    mean = jnp.sum(x, axis=1, keepdims=True) / _T
    xc = x - mean
    var = jnp.sum(xc * xc, axis=1, keepdims=True) / (_T - 1)
    denom = jnp.sqrt(var) + _EPS

    a_row = (x - mean) / denom
    b_row = (rows[1:2, :] - mean) / denom
    c_row = (rows[2:3, :] - mean) / denom

    jj = jax.lax.broadcasted_iota(jnp.int32, (_BR, _T), 1)
    ii0 = jax.lax.broadcasted_iota(jnp.int32, (_BR, _T), 0)
    col_valid = jj < _N

    xt = cols_ref[0]
    cm_vec = jnp.zeros((1, _T), jnp.float32)
    cm1_vec = jnp.zeros((1, _T), jnp.float32)
    for i0 in range(0, _T, _BR):
        a_col = (xt[i0:i0 + _BR, :] - mean) / denom
        b_col = (xt[i0 + 1:i0 + 1 + _BR, :] - mean) / denom
        c_col = (xt[i0 + 2:i0 + 2 + _BR, :] - mean) / denom
        dm = jnp.maximum(jnp.abs(a_col - a_row), jnp.abs(b_col - b_row))
        dm1 = jnp.maximum(dm, jnp.abs(c_col - c_row))
        valid = ((ii0 + i0) < _N) & col_valid
        m_hit = jnp.where(valid & (dm <= _R), 1.0, 0.0)
        m1_hit = jnp.where(valid & (dm1 <= _R), 1.0, 0.0)
        cm_vec = cm_vec + jnp.sum(m_hit, axis=0, keepdims=True)
        cm1_vec = cm1_vec + jnp.sum(m1_hit, axis=0, keepdims=True)

    cm = jnp.sum(cm_vec, axis=1, keepdims=True)
    cm1 = jnp.sum(cm1_vec, axis=1, keepdims=True)
    ratio = cm1 / jnp.maximum(cm, 1.0)
    ent = -jnp.log(jnp.maximum(ratio, 1e-30))
    ent = jnp.where((cm > 0) & (cm1 > 0), ent, 0.0)
    out_ref[...] = jnp.broadcast_to(ent, (1, 1, 128))


def _sample_entropies(X, *, interpret=False):
    S = X.shape[0]
    rows3 = jnp.stack(
        [X, jnp.roll(X, -1, axis=1), jnp.roll(X, -2, axis=1)], axis=1
    )
    cols = jnp.pad(X, ((0, 0), (0, _TPAD - _T))).reshape(S, _TPAD, 1)
    out = pl.pallas_call(
        _entropy_kernel,
        grid=(S,),
        in_specs=[
            pl.BlockSpec((1, 3, _T), lambda s: (s, 0, 0)),
            pl.BlockSpec((1, _TPAD, 1), lambda s: (s, 0, 0)),
        ],
        out_specs=pl.BlockSpec((1, 1, 128), lambda s: (s, 0, 0)),
        out_shape=jax.ShapeDtypeStruct((S, 1, 128), jnp.float32),
        compiler_params=pltpu.CompilerParams(
            dimension_semantics=("parallel",),
        ),
        name="sample_entropy",
        interpret=interpret,
    )(rows3, cols)
    return out[:, 0, 0]


def kernel(predictions, targets):
    B, C, T = predictions.shape
    X = jnp.concatenate(
        [predictions.reshape(B * C, T), targets.reshape(B * C, T)], axis=0
    )
    ents = _sample_entropies(X)
    half = X.shape[0] // 2
    return jnp.mean((ents[:half] - ents[half:]) ** 2)


# speedup vs baseline: 1.2585x; 1.0875x over previous
import jax
import jax.numpy as jnp
from jax.experimental import pallas as pl
from jax.experimental.pallas import tpu as pltpu

_M = 2
_R = 0.2
_EPS = 1e-8
_T = 1024
_N = _T - _M
_TPAD = 1032
_BR = 256


_BIG = 1e30
_PACK = 2048.0


def _entropy_kernel(rows_ref, cols_ref, out_ref):
    rows = rows_ref[0]
    x = rows[0:1, :]
    mean = jnp.sum(x, axis=1, keepdims=True) / _T
    xc = x - mean
    var = jnp.sum(xc * xc, axis=1, keepdims=True) / (_T - 1)
    inv = 1.0 / (jnp.sqrt(var) + _EPS)

    jj = jax.lax.broadcasted_iota(jnp.int32, (1, _T), 1)
    a_row = jnp.where(jj >= _N, _BIG, (x - mean) * inv)
    b_row = (rows[1:2, :] - mean) * inv
    c_row = (rows[2:3, :] - mean) * inv

    iv = jax.lax.broadcasted_iota(jnp.int32, (_BR, 1), 0)
    xt = cols_ref[0]
    acc = jnp.zeros((1, _T), jnp.float32)
    for i0 in range(0, _T, _BR):
        a_col = (xt[i0:i0 + _BR, :] - mean) * inv
        if i0 + _BR > _N:
            a_col = jnp.where(iv >= _N - i0, -_BIG, a_col)
        b_col = (xt[i0 + 1:i0 + 1 + _BR, :] - mean) * inv
        c_col = (xt[i0 + 2:i0 + 2 + _BR, :] - mean) * inv
        dm = jnp.maximum(jnp.abs(a_col - a_row), jnp.abs(b_col - b_row))
        cd = jnp.abs(c_col - c_row)
        contrib = jnp.where(dm <= _R,
                            jnp.where(cd <= _R, 1.0 + _PACK, 1.0), 0.0)
        acc = acc + jnp.sum(contrib, axis=0, keepdims=True)

    cm1_vec = jnp.floor(acc * (1.0 / _PACK))
    cm_vec = acc - _PACK * cm1_vec
    cm = jnp.sum(cm_vec, axis=1, keepdims=True)
    cm1 = jnp.sum(cm1_vec, axis=1, keepdims=True)
    ratio = cm1 / jnp.maximum(cm, 1.0)
    ent = -jnp.log(jnp.maximum(ratio, 1e-30))
    ent = jnp.where((cm > 0) & (cm1 > 0), ent, 0.0)
    out_ref[...] = jnp.broadcast_to(ent, (1, 1, 128))


def _sample_entropies(X, *, interpret=False):
    S = X.shape[0]
    rows3 = jnp.stack(
        [X, jnp.roll(X, -1, axis=1), jnp.roll(X, -2, axis=1)], axis=1
    )
    cols = jnp.pad(X, ((0, 0), (0, _TPAD - _T))).reshape(S, _TPAD, 1)
    out = pl.pallas_call(
        _entropy_kernel,
        grid=(S,),
        in_specs=[
            pl.BlockSpec((1, 3, _T), lambda s: (s, 0, 0)),
            pl.BlockSpec((1, _TPAD, 1), lambda s: (s, 0, 0)),
        ],
        out_specs=pl.BlockSpec((1, 1, 128), lambda s: (s, 0, 0)),
        out_shape=jax.ShapeDtypeStruct((S, 1, 128), jnp.float32),
        compiler_params=pltpu.CompilerParams(
            dimension_semantics=("parallel",),
        ),
        name="sample_entropy",
        interpret=interpret,
    )(rows3, cols)
    return out[:, 0, 0]


def kernel(predictions, targets):
    B, C, T = predictions.shape
    X = jnp.concatenate(
        [predictions.reshape(B * C, T), targets.reshape(B * C, T)], axis=0
    )
    ents = _sample_entropies(X)
    half = X.shape[0] // 2
    return jnp.mean((ents[:half] - ents[half:]) ** 2)


# speedup vs baseline: 4.7717x; 3.7916x over previous
import jax
import jax.numpy as jnp
from jax.experimental import pallas as pl
from jax.experimental.pallas import tpu as pltpu

_M = 2
_R = 0.2
_EPS = 1e-8
_T = 1024
_N = _T - _M
_TPAD = 1032
_BR = 256


_BIG = 1e30
_PACK = 512.0
_BC = 128


_G = 4


def _prologue(rows):
    x = rows[0:1, :]
    mean = jnp.sum(x, axis=1, keepdims=True) / _T
    xc = x - mean
    var = jnp.sum(xc * xc, axis=1, keepdims=True) / (_T - 1)
    inv = 1.0 / (jnp.sqrt(var) + _EPS)
    jj = jax.lax.broadcasted_iota(jnp.int32, (1, _T), 1)
    an = (x - mean) * inv
    a_row = jnp.where(jj >= _N, _BIG, an)
    a_colsrc = jnp.where(jj >= _N, -_BIG, an)
    b_all = (rows[1:2, :] - mean) * inv
    c_all = (rows[2:3, :] - mean) * inv
    return a_row, a_colsrc, b_all, c_all


def _entropy_kernel(rows_ref, out_ref):
    sigs = [_prologue(rows_ref[g]) for g in range(_G)]

    acc_up = [jnp.zeros((8, _BC), jnp.float32) for _ in range(_G)]
    acc_dg = [jnp.zeros((8, _BC), jnp.float32) for _ in range(_G)]
    for bi in range(_T // _BC):
        sl_i = slice(bi * _BC, (bi + 1) * _BC)
        cbs = []
        for g in range(_G):
            a_row, a_colsrc, b_all, c_all = sigs[g]
            cbs.append((
                jnp.transpose(jnp.broadcast_to(a_colsrc[:, sl_i], (_BC, _BC))),
                jnp.transpose(jnp.broadcast_to(b_all[:, sl_i], (_BC, _BC))),
                jnp.transpose(jnp.broadcast_to(c_all[:, sl_i], (_BC, _BC))),
            ))
        for bj in range(bi, _T // _BC):
            sl_j = slice(bj * _BC, (bj + 1) * _BC)
            for g in range(_G):
                a_row, a_colsrc, b_all, c_all = sigs[g]
                a_cb, b_cb, c_cb = cbs[g]
                dm = jnp.maximum(jnp.abs(a_cb - a_row[:, sl_j]),
                                 jnp.abs(b_cb - b_all[:, sl_j]))
                cd = jnp.abs(c_cb - c_all[:, sl_j])
                contrib = jnp.where(dm <= _R,
                                    jnp.where(cd <= _R, 1.0 + _PACK, 1.0), 0.0)
                t = jnp.sum(contrib.reshape(16, 8, _BC), axis=0)
                if bi == bj:
                    acc_dg[g] = acc_dg[g] + t
                else:
                    acc_up[g] = acc_up[g] + t

    for g in range(_G):
        cm1_up = jnp.floor(acc_up[g] * (1.0 / _PACK))
        cm_up = acc_up[g] - _PACK * cm1_up
        cm1_dg = jnp.floor(acc_dg[g] * (1.0 / _PACK))
        cm_dg = acc_dg[g] - _PACK * cm1_dg
        cm = (2.0 * jnp.sum(cm_up, axis=(0, 1), keepdims=True)
              + jnp.sum(cm_dg, axis=(0, 1), keepdims=True)).reshape(1, 1)
        cm1 = (2.0 * jnp.sum(cm1_up, axis=(0, 1), keepdims=True)
               + jnp.sum(cm1_dg, axis=(0, 1), keepdims=True)).reshape(1, 1)
        ratio = cm1 / jnp.maximum(cm, 1.0)
        ent = -jnp.log(jnp.maximum(ratio, 1e-30))
        ent = jnp.where((cm > 0) & (cm1 > 0), ent, 0.0)
        out_ref[g, :, :] = jnp.broadcast_to(ent, (1, 128))


def _sample_entropies(X, *, interpret=False):
    S = X.shape[0]
    rows3 = jnp.stack(
        [X, jnp.roll(X, -1, axis=1), jnp.roll(X, -2, axis=1)], axis=1
    )
    out = pl.pallas_call(
        _entropy_kernel,
        grid=(S // _G,),
        in_specs=[
            pl.BlockSpec((_G, 3, _T), lambda s: (s, 0, 0)),
        ],
        out_specs=pl.BlockSpec((_G, 1, 128), lambda s: (s, 0, 0)),
        out_shape=jax.ShapeDtypeStruct((S, 1, 128), jnp.float32),
        compiler_params=pltpu.CompilerParams(
            dimension_semantics=("parallel",),
        ),
        name="sample_entropy",
        interpret=interpret,
    )(rows3)
    return out[:, 0, 0]


def kernel(predictions, targets):
    B, C, T = predictions.shape
    X = jnp.concatenate(
        [predictions.reshape(B * C, T), targets.reshape(B * C, T)], axis=0
    )
    ents = _sample_entropies(X)
    half = X.shape[0] // 2
    return jnp.mean((ents[:half] - ents[half:]) ** 2)


# speedup vs baseline: 4.9644x; 1.0404x over previous
import jax
import jax.numpy as jnp
from jax.experimental import pallas as pl
from jax.experimental.pallas import tpu as pltpu

_M = 2
_R = 0.2
_EPS = 1e-8
_T = 1024
_N = _T - _M
_TPAD = 1032
_BR = 256


_BIG = 1e30
_PACK = 512.0
_BC = 128


_G = 8
_NCORES = 2


def _prologue(rows):
    x = rows[0:1, :]
    mean = jnp.sum(x, axis=1, keepdims=True) / _T
    xc = x - mean
    var = jnp.sum(xc * xc, axis=1, keepdims=True) / (_T - 1)
    inv = 1.0 / (jnp.sqrt(var) + _EPS)
    jj = jax.lax.broadcasted_iota(jnp.int32, (1, _T), 1)
    an = (x - mean) * inv
    a_row = jnp.where(jj >= _N, _BIG, an)
    a_colsrc = jnp.where(jj >= _N, -_BIG, an)
    b_all = (rows[1:2, :] - mean) * inv
    c_all = (rows[2:3, :] - mean) * inv
    return a_row, a_colsrc, b_all, c_all


def _entropy_kernel(rows_ref, out_ref):
    sigs = [_prologue(rows_ref[g]) for g in range(_G)]

    acc_up = [jnp.zeros((8, _BC), jnp.float32) for _ in range(_G)]
    acc_dg = [jnp.zeros((8, _BC), jnp.float32) for _ in range(_G)]
    for bi in range(_T // _BC):
        sl_i = slice(bi * _BC, (bi + 1) * _BC)
        cbs = []
        for g in range(_G):
            a_row, a_colsrc, b_all, c_all = sigs[g]
            cbs.append((
                jnp.transpose(jnp.broadcast_to(a_colsrc[:, sl_i], (_BC, _BC))),
                jnp.transpose(jnp.broadcast_to(b_all[:, sl_i], (_BC, _BC))),
                jnp.transpose(jnp.broadcast_to(c_all[:, sl_i], (_BC, _BC))),
            ))
        for bj in range(bi, _T // _BC):
            sl_j = slice(bj * _BC, (bj + 1) * _BC)
            for g in range(_G):
                a_row, a_colsrc, b_all, c_all = sigs[g]
                a_cb, b_cb, c_cb = cbs[g]
                dm = jnp.maximum(jnp.abs(a_cb - a_row[:, sl_j]),
                                 jnp.abs(b_cb - b_all[:, sl_j]))
                cd = jnp.abs(c_cb - c_all[:, sl_j])
                contrib = jnp.where(dm <= _R,
                                    jnp.where(cd <= _R, 1.0 + _PACK, 1.0), 0.0)
                t = jnp.sum(contrib.reshape(16, 8, _BC), axis=0)
                if bi == bj:
                    acc_dg[g] = acc_dg[g] + t
                else:
                    acc_up[g] = acc_up[g] + t

    for g in range(_G):
        cm1_up = jnp.floor(acc_up[g] * (1.0 / _PACK))
        cm_up = acc_up[g] - _PACK * cm1_up
        cm1_dg = jnp.floor(acc_dg[g] * (1.0 / _PACK))
        cm_dg = acc_dg[g] - _PACK * cm1_dg
        cm = (2.0 * jnp.sum(cm_up, axis=(0, 1), keepdims=True)
              + jnp.sum(cm_dg, axis=(0, 1), keepdims=True)).reshape(1, 1)
        cm1 = (2.0 * jnp.sum(cm1_up, axis=(0, 1), keepdims=True)
               + jnp.sum(cm1_dg, axis=(0, 1), keepdims=True)).reshape(1, 1)
        ratio = cm1 / jnp.maximum(cm, 1.0)
        ent = -jnp.log(jnp.maximum(ratio, 1e-30))
        ent = jnp.where((cm > 0) & (cm1 > 0), ent, 0.0)
        out_ref[g, :, :] = jnp.broadcast_to(ent, (1, 128))


def _sample_entropies(X, *, interpret=False):
    S = X.shape[0]
    rows3 = jnp.stack(
        [X, jnp.roll(X, -1, axis=1), jnp.roll(X, -2, axis=1)], axis=1
    )
    out = pl.pallas_call(
        _entropy_kernel,
        grid=(S // _G,),
        in_specs=[
            pl.BlockSpec((_G, 3, _T), lambda s: (s, 0, 0)),
        ],
        out_specs=pl.BlockSpec((_G, 1, 128), lambda s: (s, 0, 0)),
        out_shape=jax.ShapeDtypeStruct((S, 1, 128), jnp.float32),
        compiler_params=pltpu.CompilerParams(
            dimension_semantics=("parallel",),
        ),
        name="sample_entropy",
        interpret=interpret,
    )(rows3)
    return out[:, 0, 0]


def kernel(predictions, targets):
    B, C, T = predictions.shape
    X = jnp.concatenate(
        [predictions.reshape(B * C, T), targets.reshape(B * C, T)], axis=0
    )
    ents = _sample_entropies(X)
    half = X.shape[0] // 2
    return jnp.mean((ents[:half] - ents[half:]) ** 2)


# speedup vs baseline: 5.2074x; 1.0489x over previous
import jax
import jax.numpy as jnp
from jax.experimental import pallas as pl
from jax.experimental.pallas import tpu as pltpu

_M = 2
_R = 0.2
_EPS = 1e-8
_T = 1024
_N = _T - _M
_TPAD = 1032
_BR = 256


_BIG = 1e30
_PACK = 4096.0
_BC = 128


_G = 16
_NCORES = 2


def _prologue(rows):
    x = rows[0:1, :]
    mean = jnp.sum(x, axis=1, keepdims=True) / _T
    xc = x - mean
    var = jnp.sum(xc * xc, axis=1, keepdims=True) / (_T - 1)
    inv = 1.0 / (jnp.sqrt(var) + _EPS)
    jj = jax.lax.broadcasted_iota(jnp.int32, (1, _T), 1)
    an = (x - mean) * inv
    a_row = jnp.where(jj >= _N, _BIG, an)
    a_colsrc = jnp.where(jj >= _N, -_BIG, an)
    b_all = (rows[1:2, :] - mean) * inv
    c_all = (rows[2:3, :] - mean) * inv
    return a_row, a_colsrc, b_all, c_all


def _entropy_kernel(rows_ref, out_ref):
    sigs = [_prologue(rows_ref[g]) for g in range(_G)]

    ones_row = jnp.ones((1, _BC), jnp.float32)
    acc_up = [jnp.zeros((1, _BC), jnp.float32) for _ in range(_G)]
    acc_dg = [jnp.zeros((1, _BC), jnp.float32) for _ in range(_G)]
    for bi in range(_T // _BC):
        sl_i = slice(bi * _BC, (bi + 1) * _BC)
        cbs = []
        for g in range(_G):
            a_row, a_colsrc, b_all, c_all = sigs[g]
            cbs.append((
                jnp.transpose(jnp.broadcast_to(a_colsrc[:, sl_i], (_BC, _BC))),
                jnp.transpose(jnp.broadcast_to(b_all[:, sl_i], (_BC, _BC))),
                jnp.transpose(jnp.broadcast_to(c_all[:, sl_i], (_BC, _BC))),
            ))
        for bj in range(bi, _T // _BC):
            sl_j = slice(bj * _BC, (bj + 1) * _BC)
            for g in range(_G):
                a_row, a_colsrc, b_all, c_all = sigs[g]
                a_cb, b_cb, c_cb = cbs[g]
                dm = jnp.maximum(jnp.abs(a_cb - a_row[:, sl_j]),
                                 jnp.abs(b_cb - b_all[:, sl_j]))
                cd = jnp.abs(c_cb - c_all[:, sl_j])
                contrib = jnp.where(dm <= _R,
                                    jnp.where(cd <= _R, 1.0 + _PACK, 1.0), 0.0)
                t = jnp.dot(ones_row, contrib,
                            preferred_element_type=jnp.float32)
                if bi == bj:
                    acc_dg[g] = acc_dg[g] + t
                else:
                    acc_up[g] = acc_up[g] + t

    for g in range(_G):
        cm1_up = jnp.floor(acc_up[g] * (1.0 / _PACK))
        cm_up = acc_up[g] - _PACK * cm1_up
        cm1_dg = jnp.floor(acc_dg[g] * (1.0 / _PACK))
        cm_dg = acc_dg[g] - _PACK * cm1_dg
        cm = (2.0 * jnp.sum(cm_up, axis=1, keepdims=True)
              + jnp.sum(cm_dg, axis=1, keepdims=True))
        cm1 = (2.0 * jnp.sum(cm1_up, axis=1, keepdims=True)
               + jnp.sum(cm1_dg, axis=1, keepdims=True))
        ratio = cm1 / jnp.maximum(cm, 1.0)
        ent = -jnp.log(jnp.maximum(ratio, 1e-30))
        ent = jnp.where((cm > 0) & (cm1 > 0), ent, 0.0)
        out_ref[g, :, :] = jnp.broadcast_to(ent, (1, 128))


def _sample_entropies(X, *, interpret=False):
    S = X.shape[0]
    rows3 = jnp.stack(
        [X, jnp.roll(X, -1, axis=1), jnp.roll(X, -2, axis=1)], axis=1
    )
    out = pl.pallas_call(
        _entropy_kernel,
        grid=(S // _G,),
        in_specs=[
            pl.BlockSpec((_G, 3, _T), lambda s: (s, 0, 0)),
        ],
        out_specs=pl.BlockSpec((_G, 1, 128), lambda s: (s, 0, 0)),
        out_shape=jax.ShapeDtypeStruct((S, 1, 128), jnp.float32),
        compiler_params=pltpu.CompilerParams(
            dimension_semantics=("parallel",),
        ),
        name="sample_entropy",
        interpret=interpret,
    )(rows3)
    return out[:, 0, 0]


def kernel(predictions, targets):
    B, C, T = predictions.shape
    X = jnp.concatenate(
        [predictions.reshape(B * C, T), targets.reshape(B * C, T)], axis=0
    )
    ents = _sample_entropies(X)
    half = X.shape[0] // 2
    return jnp.mean((ents[:half] - ents[half:]) ** 2)


# speedup vs baseline: 5.2882x; 1.0155x over previous
import jax
import jax.numpy as jnp
from jax.experimental import pallas as pl
from jax.experimental.pallas import tpu as pltpu

_M = 2
_R = 0.2
_EPS = 1e-8
_T = 1024
_N = _T - _M
_TPAD = 1032
_BR = 256


_BIG = 1e30
_PACK = 4096.0
_BC = 128


_G = 16
_NCORES = 2


def _prologue(rows):
    x = rows[0:1, :]
    mean = jnp.sum(x, axis=1, keepdims=True) / _T
    xc = x - mean
    var = jnp.sum(xc * xc, axis=1, keepdims=True) / (_T - 1)
    inv = 1.0 / (jnp.sqrt(var) + _EPS)
    jj = jax.lax.broadcasted_iota(jnp.int32, (1, _T), 1)
    an = (x - mean) * inv
    a_row = jnp.where(jj >= _N, _BIG, an)
    a_colsrc = jnp.where(jj >= _N, -_BIG, an)
    b_all = (rows[1:2, :] - mean) * inv
    c_all = (rows[2:3, :] - mean) * inv
    return a_row, a_colsrc, b_all, c_all


def _entropy_kernel(rows_ref, out_ref):
    sigs = [_prologue(rows_ref[g]) for g in range(_G)]

    ones_row = jnp.ones((1, _BC), jnp.float32)
    zeros = jnp.zeros((1, _BC), jnp.float32)
    acc = [[zeros, zeros, zeros, zeros] for _ in range(_G)]
    for bi in range(_T // _BC):
        sl_i = slice(bi * _BC, (bi + 1) * _BC)
        cbs = []
        for g in range(_G):
            a_row, a_colsrc, b_all, c_all = sigs[g]
            cbs.append((
                jnp.transpose(jnp.broadcast_to(a_colsrc[:, sl_i], (_BC, _BC))),
                jnp.transpose(jnp.broadcast_to(b_all[:, sl_i], (_BC, _BC))),
                jnp.transpose(jnp.broadcast_to(c_all[:, sl_i], (_BC, _BC))),
            ))
        for bj in range(bi, _T // _BC):
            sl_j = slice(bj * _BC, (bj + 1) * _BC)
            for g in range(_G):
                a_row, a_colsrc, b_all, c_all = sigs[g]
                a_cb, b_cb, c_cb = cbs[g]
                dm = jnp.maximum(jnp.abs(a_cb - a_row[:, sl_j]),
                                 jnp.abs(b_cb - b_all[:, sl_j]))
                cd = jnp.abs(c_cb - c_all[:, sl_j])
                mask_m = dm <= _R
                h_m = jnp.where(mask_m, 1.0, 0.0)
                h_m1 = jnp.where(mask_m & (cd <= _R), 1.0, 0.0)
                t_m = jnp.dot(ones_row, h_m,
                              preferred_element_type=jnp.float32)
                t_m1 = jnp.dot(ones_row, h_m1,
                               preferred_element_type=jnp.float32)
                o = 0 if bi != bj else 2
                acc[g][o] = acc[g][o] + t_m
                acc[g][o + 1] = acc[g][o + 1] + t_m1

    for g in range(_G):
        up_m, up_m1, dg_m, dg_m1 = acc[g]
        cm = (2.0 * jnp.sum(up_m, axis=1, keepdims=True)
              + jnp.sum(dg_m, axis=1, keepdims=True))
        cm1 = (2.0 * jnp.sum(up_m1, axis=1, keepdims=True)
               + jnp.sum(dg_m1, axis=1, keepdims=True))
        ratio = cm1 / jnp.maximum(cm, 1.0)
        ent = -jnp.log(jnp.maximum(ratio, 1e-30))
        ent = jnp.where((cm > 0) & (cm1 > 0), ent, 0.0)
        out_ref[g, :, :] = jnp.broadcast_to(ent, (1, 128))


def _sample_entropies(X, *, interpret=False):
    S = X.shape[0]
    rows3 = jnp.stack(
        [X, jnp.roll(X, -1, axis=1), jnp.roll(X, -2, axis=1)], axis=1
    )
    out = pl.pallas_call(
        _entropy_kernel,
        grid=(S // _G,),
        in_specs=[
            pl.BlockSpec((_G, 3, _T), lambda s: (s, 0, 0)),
        ],
        out_specs=pl.BlockSpec((_G, 1, 128), lambda s: (s, 0, 0)),
        out_shape=jax.ShapeDtypeStruct((S, 1, 128), jnp.float32),
        compiler_params=pltpu.CompilerParams(
            dimension_semantics=("parallel",),
        ),
        name="sample_entropy",
        interpret=interpret,
    )(rows3)
    return out[:, 0, 0]


def kernel(predictions, targets):
    B, C, T = predictions.shape
    X = jnp.concatenate(
        [predictions.reshape(B * C, T), targets.reshape(B * C, T)], axis=0
    )
    ents = _sample_entropies(X)
    half = X.shape[0] // 2
    return jnp.mean((ents[:half] - ents[half:]) ** 2)


# speedup vs baseline: 5.3093x; 1.0040x over previous
import jax
import jax.numpy as jnp
from jax.experimental import pallas as pl
from jax.experimental.pallas import tpu as pltpu

_M = 2
_R = 0.2
_EPS = 1e-8
_T = 1024
_N = _T - _M
_TPAD = 1032
_BR = 256


_BIG = 1e30
_PACK = 4096.0
_BC = 128


_G = 8
_NCORES = 2


def _prologue(rows):
    x = rows[0:1, :]
    mean = jnp.sum(x, axis=1, keepdims=True) / _T
    xc = x - mean
    var = jnp.sum(xc * xc, axis=1, keepdims=True) / (_T - 1)
    inv = 1.0 / (jnp.sqrt(var) + _EPS)
    jj = jax.lax.broadcasted_iota(jnp.int32, (1, _T), 1)
    an = (x - mean) * inv
    a_row = jnp.where(jj >= _N, _BIG, an)
    a_colsrc = jnp.where(jj >= _N, -_BIG, an)
    b_all = (rows[1:2, :] - mean) * inv
    c_all = (rows[2:3, :] - mean) * inv
    return a_row, a_colsrc, b_all, c_all


def _entropy_kernel(rows_ref, out_ref):
    sigs = [_prologue(rows_ref[g]) for g in range(_G)]

    ones_row = jnp.ones((1, _BC), jnp.float32)
    zeros = jnp.zeros((1, _BC), jnp.float32)
    acc = [[zeros, zeros, zeros, zeros] for _ in range(_G)]
    for bi in range(_T // _BC):
        sl_i = slice(bi * _BC, (bi + 1) * _BC)
        cbs = []
        for g in range(_G):
            a_row, a_colsrc, b_all, c_all = sigs[g]
            cbs.append((
                jnp.transpose(jnp.broadcast_to(a_colsrc[:, sl_i], (_BC, _BC))),
                jnp.transpose(jnp.broadcast_to(b_all[:, sl_i], (_BC, _BC))),
                jnp.transpose(jnp.broadcast_to(c_all[:, sl_i], (_BC, _BC))),
            ))
        for bj in range(bi, _T // _BC):
            sl_j = slice(bj * _BC, (bj + 1) * _BC)
            for g in range(_G):
                a_row, a_colsrc, b_all, c_all = sigs[g]
                a_cb, b_cb, c_cb = cbs[g]
                dm = jnp.maximum(jnp.abs(a_cb - a_row[:, sl_j]),
                                 jnp.abs(b_cb - b_all[:, sl_j]))
                cd = jnp.abs(c_cb - c_all[:, sl_j])
                mask_m = dm <= _R
                h_m = jnp.where(mask_m, 1.0, 0.0)
                h_m1 = jnp.where(mask_m & (cd <= _R), 1.0, 0.0)
                t_m = jnp.dot(ones_row, h_m,
                              preferred_element_type=jnp.float32)
                t_m1 = jnp.dot(ones_row, h_m1,
                               preferred_element_type=jnp.float32)
                o = 0 if bi != bj else 2
                acc[g][o] = acc[g][o] + t_m
                acc[g][o + 1] = acc[g][o + 1] + t_m1

    for g in range(_G):
        up_m, up_m1, dg_m, dg_m1 = acc[g]
        cm = (2.0 * jnp.sum(up_m, axis=1, keepdims=True)
              + jnp.sum(dg_m, axis=1, keepdims=True))
        cm1 = (2.0 * jnp.sum(up_m1, axis=1, keepdims=True)
               + jnp.sum(dg_m1, axis=1, keepdims=True))
        ratio = cm1 / jnp.maximum(cm, 1.0)
        ent = -jnp.log(jnp.maximum(ratio, 1e-30))
        ent = jnp.where((cm > 0) & (cm1 > 0), ent, 0.0)
        out_ref[g, :, :] = jnp.broadcast_to(ent, (1, 128))


def _sample_entropies(X, *, interpret=False):
    S = X.shape[0]
    rows3 = jnp.stack(
        [X, jnp.roll(X, -1, axis=1), jnp.roll(X, -2, axis=1)], axis=1
    )
    out = pl.pallas_call(
        _entropy_kernel,
        grid=(S // _G,),
        in_specs=[
            pl.BlockSpec((_G, 3, _T), lambda s: (s, 0, 0)),
        ],
        out_specs=pl.BlockSpec((_G, 1, 128), lambda s: (s, 0, 0)),
        out_shape=jax.ShapeDtypeStruct((S, 1, 128), jnp.float32),
        compiler_params=pltpu.CompilerParams(
            dimension_semantics=("parallel",),
        ),
        name="sample_entropy",
        interpret=interpret,
    )(rows3)
    return out[:, 0, 0]


def kernel(predictions, targets):
    B, C, T = predictions.shape
    X = jnp.concatenate(
        [predictions.reshape(B * C, T), targets.reshape(B * C, T)], axis=0
    )
    ents = _sample_entropies(X)
    half = X.shape[0] // 2
    return jnp.mean((ents[:half] - ents[half:]) ** 2)


# speedup vs baseline: 5.6275x; 1.0599x over previous
import jax
import jax.numpy as jnp
from jax.experimental import pallas as pl
from jax.experimental.pallas import tpu as pltpu

_M = 2
_R = 0.2
_EPS = 1e-8
_T = 1024
_N = _T - _M
_TPAD = 1032
_BR = 256


_BIG = 1e30
_PACK = 4096.0
_BC = 128


_G = 8
_NCORES = 2


def _prologue(rows):
    x = rows[0:1, :]
    mean = jnp.sum(x, axis=1, keepdims=True) / _T
    xc = x - mean
    var = jnp.sum(xc * xc, axis=1, keepdims=True) / (_T - 1)
    inv = 1.0 / (jnp.sqrt(var) + _EPS)
    jj = jax.lax.broadcasted_iota(jnp.int32, (1, _T), 1)
    an = (x - mean) * inv
    a_row = jnp.where(jj >= _N, _BIG, an)
    a_colsrc = jnp.where(jj >= _N, -_BIG, an)
    b_all = (rows[1:2, :] - mean) * inv
    c_all = (rows[2:3, :] - mean) * inv
    return a_row, a_colsrc, b_all, c_all


def _entropy_kernel(rows_ref, out_ref):
    sigs = [_prologue(rows_ref[g]) for g in range(_G)]

    ones_row = jnp.ones((1, _BC), jnp.float32)
    zeros = jnp.zeros((1, _BC), jnp.float32)
    acc = [[zeros, zeros, zeros, zeros] for _ in range(_G)]
    for bi in range(_T // _BC):
        sl_i = slice(bi * _BC, (bi + 1) * _BC)
        cbs = []
        for g in range(_G):
            a_row, a_colsrc, b_all, c_all = sigs[g]
            cbs.append((
                jnp.transpose(jnp.broadcast_to(a_colsrc[:, sl_i], (_BC, _BC))),
                jnp.transpose(jnp.broadcast_to(b_all[:, sl_i], (_BC, _BC))),
                jnp.transpose(jnp.broadcast_to(c_all[:, sl_i], (_BC, _BC))),
            ))
        for bj in range(bi, _T // _BC):
            sl_j = slice(bj * _BC, (bj + 1) * _BC)
            for g in range(_G):
                a_row, a_colsrc, b_all, c_all = sigs[g]
                a_cb, b_cb, c_cb = cbs[g]
                dm = jnp.maximum(jnp.abs(a_cb - a_row[:, sl_j]),
                                 jnp.abs(b_cb - b_all[:, sl_j]))
                cd = jnp.abs(c_cb - c_all[:, sl_j])
                mask_m = dm <= _R
                h_m = jnp.where(mask_m, 1.0, 0.0)
                h_m1 = jnp.where(mask_m & (cd <= _R), 1.0, 0.0)
                t_m = jnp.dot(ones_row, h_m,
                              preferred_element_type=jnp.float32)
                t_m1 = jnp.dot(ones_row, h_m1,
                               preferred_element_type=jnp.float32)
                o = 0 if bi != bj else 2
                acc[g][o] = acc[g][o] + t_m
                acc[g][o + 1] = acc[g][o + 1] + t_m1

    for g in range(_G):
        up_m, up_m1, dg_m, dg_m1 = acc[g]
        cm = (2.0 * jnp.sum(up_m, axis=1, keepdims=True)
              + jnp.sum(dg_m, axis=1, keepdims=True))
        cm1 = (2.0 * jnp.sum(up_m1, axis=1, keepdims=True)
               + jnp.sum(dg_m1, axis=1, keepdims=True))
        ratio = cm1 / jnp.maximum(cm, 1.0)
        ent = -jnp.log(jnp.maximum(ratio, 1e-30))
        ent = jnp.where((cm > 0) & (cm1 > 0), ent, 0.0)
        out_ref[g, :, :] = jnp.broadcast_to(ent, (1, 128))


def _sample_entropies(X, *, interpret=False):
    S = X.shape[0]
    rows3 = jnp.stack(
        [X, jnp.roll(X, -1, axis=1), jnp.roll(X, -2, axis=1)], axis=1
    )
    out = pl.pallas_call(
        _entropy_kernel,
        grid=(S // _G,),
        in_specs=[
            pl.BlockSpec((_G, 3, _T), lambda s: (s, 0, 0)),
        ],
        out_specs=pl.BlockSpec((_G, 1, 128), lambda s: (s, 0, 0)),
        out_shape=jax.ShapeDtypeStruct((S, 1, 128), jnp.float32),
        compiler_params=pltpu.CompilerParams(
            dimension_semantics=("parallel",),
            allow_input_fusion=[True],
        ),
        name="sample_entropy",
        interpret=interpret,
    )(rows3)
    return out[:, 0, 0]


def kernel(predictions, targets):
    B, C, T = predictions.shape
    X = jnp.concatenate(
        [predictions.reshape(B * C, T), targets.reshape(B * C, T)], axis=0
    )
    ents = _sample_entropies(X)
    half = X.shape[0] // 2
    return jnp.mean((ents[:half] - ents[half:]) ** 2)
